# baseline (speedup 1.0000x reference)
import jax
import jax.numpy as jnp
import numpy as np
from jax.experimental import pallas as pl
from jax.experimental.pallas import tpu as pltpu

D_MODEL = 1024
N_HEADS = 16
D_PHI = 64
D_HEAD = 64
M_SETS = 16
N_CAND = 4
HM = N_HEADS * M_SETS
MIN_TEMP = 0.5
SC_SCALE = 1.0 / np.sqrt(D_PHI)
NEG = -1e30

S_TILE = 1024


def _group_allreduce(x, op):
    lane = jax.lax.broadcasted_iota(jnp.int32, x.shape, 1)
    for k in (1, 2, 4, 8):
        left = pltpu.roll(x, HM - k, axis=1)
        right = pltpu.roll(x, k, axis=1)
        partner = jnp.where((lane & k) == 0, left, right)
        x = op(x, partner)
    return x


def _main_kernel(x_ref, wq_ref, bq_ref, wk_ref, bk_ref, desc_ref, set_ref,
                 t2s_ref, temp_ref,
                 repr_ref, w_ref, i1_ref, i2_ref, bank_ref,
                 kbd_ref, sbd_ref):
    @pl.when(pl.program_id(1) == 0)
    def _prologue():
        k = jnp.dot(desc_ref[0], wk_ref[...],
                    preferred_element_type=jnp.float32) + bk_ref[...]
        rowblocks = []
        for h in range(N_HEADS):
            k_h_t = jnp.transpose(k[:, h * D_PHI:(h + 1) * D_PHI])
            pads = []
            if h > 0:
                pads.append(jnp.zeros((D_PHI, h * M_SETS), jnp.float32))
            pads.append(k_h_t)
            if h < N_HEADS - 1:
                pads.append(jnp.zeros((D_PHI, HM - (h + 1) * M_SETS),
                                      jnp.float32))
            rowblocks.append(jnp.concatenate(pads, axis=1))
        kbd_ref[...] = jnp.concatenate(rowblocks, axis=0)

        set_b = set_ref[0]
        col_h = jax.lax.broadcasted_iota(jnp.int32, (M_SETS, D_MODEL),
                                         1) >> 6
        blocks = [jnp.where(col_h == h, set_b, 0.0) for h in range(N_HEADS)]
        sbd_ref[...] = jnp.concatenate(blocks, axis=0)

    q = jnp.dot(x_ref[0], wq_ref[...],
                preferred_element_type=jnp.float32) + bq_ref[...]
    scores = jnp.dot(q, kbd_ref[...],
                     preferred_element_type=jnp.float32) * SC_SCALE

    s_t = scores.shape[0]
    lane = jax.lax.broadcasted_iota(jnp.int32, (s_t, HM), 1)
    lane_m = lane & 15

    mask = jnp.zeros((s_t, HM), dtype=jnp.bool_)
    for c in range(N_CAND):
        mask = mask | (t2s_ref[:, c:c + 1] == lane_m)
    masked = jnp.where(mask, scores, NEG)

    s_int = jax.lax.bitcast_convert_type(masked, jnp.int32)
    key = jnp.where(s_int < 0, s_int ^ jnp.int32(0x7FFFFFFF), s_int)
    keym = (key & jnp.int32(-16)) | (15 - lane_m)

    k1 = _group_allreduce(keym, jnp.maximum)
    idx1 = 15 - (k1 & 15)
    m1i = jnp.where(k1 < 0, k1 ^ jnp.int32(0x7FFFFFFF), k1)
    m1 = jax.lax.bitcast_convert_type(m1i, jnp.float32)

    keym2 = jnp.where(lane_m == idx1, jnp.int32(-2147483647), keym)
    k2 = _group_allreduce(keym2, jnp.maximum)
    idx2 = 15 - (k2 & 15)

    sel = (lane_m == idx1) | (lane_m == idx2)
    tinv = 1.0 / jnp.maximum(temp_ref[0, 0], MIN_TEMP)
    w_un = jnp.where(sel, jnp.exp((masked - m1) * tinv), 0.0)

    lane_i = jax.lax.broadcasted_iota(jnp.int32, (HM, HM), 0)
    lane_j = jax.lax.broadcasted_iota(jnp.int32, (HM, HM), 1)
    gmat = ((lane_i >> 4) == (lane_j >> 4)).astype(jnp.float32)
    denom = jnp.dot(w_un, gmat, preferred_element_type=jnp.float32,
                    precision=jax.lax.Precision.HIGHEST)
    weights = w_un / denom
    for h in range(N_HEADS):
        w_ref[0, h] = weights[:, h * M_SETS:(h + 1) * M_SETS]

    qi = jax.lax.broadcasted_iota(jnp.int32, (HM, M_SETS), 0)
    qj = jax.lax.broadcasted_iota(jnp.int32, (HM, M_SETS), 1)
    qmat = ((qi >> 4) == qj).astype(jnp.float32)
    pmat = ((qi & 15) == qj).astype(jnp.float32)

    start = (lane & 15) == 0
    i1c = jnp.where(start, idx1.astype(jnp.float32), 0.0)
    i2c = jnp.where(start, idx2.astype(jnp.float32), 0.0)
    i1_ref[0] = jnp.dot(i1c, qmat,
                        preferred_element_type=jnp.float32).astype(jnp.int32)
    i2_ref[0] = jnp.dot(i2c, qmat,
                        preferred_element_type=jnp.float32).astype(jnp.int32)

    wsum = jnp.dot(weights, pmat, preferred_element_type=jnp.float32,
                   precision=jax.lax.Precision.HIGHEST)
    bmax = jnp.max(wsum, axis=1, keepdims=True)
    lane16 = jax.lax.broadcasted_iota(jnp.int32, (s_t, M_SETS), 1)
    bank = jnp.min(jnp.where(wsum == bmax, lane16, M_SETS), axis=1,
                   keepdims=True)
    bank_ref[0] = bank

    repr_ref[0] = jnp.dot(weights, sbd_ref[...],
                          preferred_element_type=jnp.float32)


def kernel(token_states, set_states, desc_router, token_to_sets,
           Wq, bq, Wk, bk, temperature):
    b, s, _ = token_states.shape

    n_tiles = s // S_TILE
    token_repr, weights, i1, i2, bank = pl.pallas_call(
        _main_kernel,
        grid=(b, n_tiles),
        in_specs=[
            pl.BlockSpec((1, S_TILE, D_MODEL), lambda i, j: (i, j, 0)),
            pl.BlockSpec((D_MODEL, D_MODEL), lambda i, j: (0, 0)),
            pl.BlockSpec((1, D_MODEL), lambda i, j: (0, 0)),
            pl.BlockSpec((D_MODEL, D_MODEL), lambda i, j: (0, 0)),
            pl.BlockSpec((1, D_MODEL), lambda i, j: (0, 0)),
            pl.BlockSpec((1, M_SETS, D_MODEL), lambda i, j: (i, 0, 0)),
            pl.BlockSpec((1, M_SETS, D_MODEL), lambda i, j: (i, 0, 0)),
            pl.BlockSpec((S_TILE, N_CAND), lambda i, j: (j, 0)),
            pl.BlockSpec((1, 1), lambda i, j: (0, 0)),
        ],
        out_specs=[
            pl.BlockSpec((1, S_TILE, D_MODEL), lambda i, j: (i, j, 0)),
            pl.BlockSpec((1, N_HEADS, S_TILE, M_SETS),
                         lambda i, j: (i, 0, j, 0)),
            pl.BlockSpec((1, S_TILE, M_SETS), lambda i, j: (i, j, 0)),
            pl.BlockSpec((1, S_TILE, M_SETS), lambda i, j: (i, j, 0)),
            pl.BlockSpec((1, S_TILE, 1), lambda i, j: (i, j, 0)),
        ],
        out_shape=[
            jax.ShapeDtypeStruct((b, s, D_MODEL), jnp.float32),
            jax.ShapeDtypeStruct((b, N_HEADS, s, M_SETS), jnp.float32),
            jax.ShapeDtypeStruct((b, s, M_SETS), jnp.int32),
            jax.ShapeDtypeStruct((b, s, M_SETS), jnp.int32),
            jax.ShapeDtypeStruct((b, s, 1), jnp.int32),
        ],
        scratch_shapes=[
            pltpu.VMEM((D_MODEL, HM), jnp.float32),
            pltpu.VMEM((HM, D_MODEL), jnp.float32),
        ],
        compiler_params=pltpu.CompilerParams(
            dimension_semantics=("arbitrary", "arbitrary")),
    )(token_states, Wq, bq.reshape(1, D_MODEL), Wk, bk.reshape(1, D_MODEL),
      desc_router, set_states, token_to_sets, temperature.reshape(1, 1))

    topk_idx = jnp.stack([i1, i2], axis=-1).transpose(0, 2, 1, 3)
    bank_indices = bank.reshape(b, s)
    return (token_repr, bank_indices, weights, topk_idx)


# speedup vs baseline: 1.3335x; 1.3335x over previous
import jax
import jax.numpy as jnp
import numpy as np
from jax.experimental import pallas as pl
from jax.experimental.pallas import tpu as pltpu

D_MODEL = 1024
N_HEADS = 16
D_PHI = 64
D_HEAD = 64
M_SETS = 16
N_CAND = 4
HM = N_HEADS * M_SETS
MIN_TEMP = 0.5
SC_SCALE = 1.0 / np.sqrt(D_PHI)
NEG = -1e30

S_TILE = 1024


def _group_allreduce(x, op):
    lane = jax.lax.broadcasted_iota(jnp.int32, x.shape, 1)
    for k in (1, 2, 4, 8):
        left = pltpu.roll(x, HM - k, axis=1)
        right = pltpu.roll(x, k, axis=1)
        partner = jnp.where((lane & k) == 0, left, right)
        x = op(x, partner)
    return x


def _main_kernel(x_ref, wq_ref, bq_ref, wk_ref, bk_ref, desc_ref, set_ref,
                 t2s_ref, temp_ref,
                 repr_ref, w_ref, i1_ref, i2_ref, bank_ref,
                 kbd_ref, sbd_ref):
    @pl.when(pl.program_id(1) == 0)
    def _prologue():
        k = jnp.dot(desc_ref[0], wk_ref[...],
                    preferred_element_type=jnp.float32) + bk_ref[...]
        rowblocks = []
        for h in range(N_HEADS):
            k_h_t = jnp.transpose(k[:, h * D_PHI:(h + 1) * D_PHI])
            pads = []
            if h > 0:
                pads.append(jnp.zeros((D_PHI, h * M_SETS), jnp.float32))
            pads.append(k_h_t)
            if h < N_HEADS - 1:
                pads.append(jnp.zeros((D_PHI, HM - (h + 1) * M_SETS),
                                      jnp.float32))
            rowblocks.append(jnp.concatenate(pads, axis=1))
        kbd_ref[...] = jnp.concatenate(rowblocks, axis=0)

        set_b = set_ref[0]
        col_h = jax.lax.broadcasted_iota(jnp.int32, (M_SETS, D_MODEL),
                                         1) >> 6
        blocks = [jnp.where(col_h == h, set_b, 0.0) for h in range(N_HEADS)]
        sbd_ref[...] = jnp.concatenate(blocks, axis=0)

    q = jnp.dot(x_ref[0], wq_ref[...],
                preferred_element_type=jnp.float32) + bq_ref[...]
    scores = jnp.dot(q, kbd_ref[...],
                     preferred_element_type=jnp.float32) * SC_SCALE

    s_t = scores.shape[0]
    lane = jax.lax.broadcasted_iota(jnp.int32, (s_t, HM), 1)
    lane_m = lane & 15

    mask = jnp.zeros((s_t, HM), dtype=jnp.bool_)
    for c in range(N_CAND):
        mask = mask | (t2s_ref[:, c:c + 1] == lane_m)
    masked = jnp.where(mask, scores, NEG)

    s_int = jax.lax.bitcast_convert_type(masked, jnp.int32)
    key = jnp.where(s_int < 0, s_int ^ jnp.int32(0x7FFFFFFF), s_int)
    keym = (key & jnp.int32(-16)) | (15 - lane_m)

    k1 = _group_allreduce(keym, jnp.maximum)
    idx1 = 15 - (k1 & 15)
    m1i = jnp.where(k1 < 0, k1 ^ jnp.int32(0x7FFFFFFF), k1)
    m1 = jax.lax.bitcast_convert_type(m1i, jnp.float32)

    keym2 = jnp.where(lane_m == idx1, jnp.int32(-2147483647), keym)
    k2 = _group_allreduce(keym2, jnp.maximum)
    idx2 = 15 - (k2 & 15)

    sel = (lane_m == idx1) | (lane_m == idx2)
    tinv = 1.0 / jnp.maximum(temp_ref[0, 0], MIN_TEMP)
    w_un = jnp.where(sel, jnp.exp((masked - m1) * tinv), 0.0)

    lane_i = jax.lax.broadcasted_iota(jnp.int32, (HM, HM), 0)
    lane_j = jax.lax.broadcasted_iota(jnp.int32, (HM, HM), 1)
    gmat = ((lane_i >> 4) == (lane_j >> 4)).astype(jnp.float32)
    denom = jnp.dot(w_un, gmat, preferred_element_type=jnp.float32,
                    precision=jax.lax.Precision.HIGHEST)
    weights = w_un / denom
    w_ref[0] = weights

    qi = jax.lax.broadcasted_iota(jnp.int32, (HM, M_SETS), 0)
    qj = jax.lax.broadcasted_iota(jnp.int32, (HM, M_SETS), 1)
    qmat = ((qi >> 4) == qj).astype(jnp.float32)
    pmat = ((qi & 15) == qj).astype(jnp.float32)

    start = (lane & 15) == 0
    i1c = jnp.where(start, idx1.astype(jnp.float32), 0.0)
    i2c = jnp.where(start, idx2.astype(jnp.float32), 0.0)
    i1_ref[0] = jnp.dot(i1c, qmat,
                        preferred_element_type=jnp.float32).astype(jnp.int32)
    i2_ref[0] = jnp.dot(i2c, qmat,
                        preferred_element_type=jnp.float32).astype(jnp.int32)

    wsum = jnp.dot(weights, pmat, preferred_element_type=jnp.float32,
                   precision=jax.lax.Precision.HIGHEST)
    bmax = jnp.max(wsum, axis=1, keepdims=True)
    lane16 = jax.lax.broadcasted_iota(jnp.int32, (s_t, M_SETS), 1)
    bank = jnp.min(jnp.where(wsum == bmax, lane16, M_SETS), axis=1,
                   keepdims=True)
    bank_ref[0] = bank

    repr_ref[0] = jnp.dot(weights, sbd_ref[...],
                          preferred_element_type=jnp.float32)


def kernel(token_states, set_states, desc_router, token_to_sets,
           Wq, bq, Wk, bk, temperature):
    b, s, _ = token_states.shape

    n_tiles = s // S_TILE
    token_repr, w_flat, i1, i2, bank = pl.pallas_call(
        _main_kernel,
        grid=(b, n_tiles),
        in_specs=[
            pl.BlockSpec((1, S_TILE, D_MODEL), lambda i, j: (i, j, 0)),
            pl.BlockSpec((D_MODEL, D_MODEL), lambda i, j: (0, 0)),
            pl.BlockSpec((1, D_MODEL), lambda i, j: (0, 0)),
            pl.BlockSpec((D_MODEL, D_MODEL), lambda i, j: (0, 0)),
            pl.BlockSpec((1, D_MODEL), lambda i, j: (0, 0)),
            pl.BlockSpec((1, M_SETS, D_MODEL), lambda i, j: (i, 0, 0)),
            pl.BlockSpec((1, M_SETS, D_MODEL), lambda i, j: (i, 0, 0)),
            pl.BlockSpec((S_TILE, N_CAND), lambda i, j: (j, 0)),
            pl.BlockSpec((1, 1), lambda i, j: (0, 0)),
        ],
        out_specs=[
            pl.BlockSpec((1, S_TILE, D_MODEL), lambda i, j: (i, j, 0)),
            pl.BlockSpec((1, S_TILE, HM), lambda i, j: (i, j, 0)),
            pl.BlockSpec((1, S_TILE, M_SETS), lambda i, j: (i, j, 0)),
            pl.BlockSpec((1, S_TILE, M_SETS), lambda i, j: (i, j, 0)),
            pl.BlockSpec((1, S_TILE, 1), lambda i, j: (i, j, 0)),
        ],
        out_shape=[
            jax.ShapeDtypeStruct((b, s, D_MODEL), jnp.float32),
            jax.ShapeDtypeStruct((b, s, HM), jnp.float32),
            jax.ShapeDtypeStruct((b, s, M_SETS), jnp.int32),
            jax.ShapeDtypeStruct((b, s, M_SETS), jnp.int32),
            jax.ShapeDtypeStruct((b, s, 1), jnp.int32),
        ],
        scratch_shapes=[
            pltpu.VMEM((D_MODEL, HM), jnp.float32),
            pltpu.VMEM((HM, D_MODEL), jnp.float32),
        ],
        compiler_params=pltpu.CompilerParams(
            dimension_semantics=("arbitrary", "arbitrary")),
    )(token_states, Wq, bq.reshape(1, D_MODEL), Wk, bk.reshape(1, D_MODEL),
      desc_router, set_states, token_to_sets, temperature.reshape(1, 1))

    weights = w_flat.reshape(b, s, N_HEADS, M_SETS).transpose(0, 2, 1, 3)
    topk_idx = jnp.stack([i1, i2], axis=-1).transpose(0, 2, 1, 3)
    bank_indices = bank.reshape(b, s)
    return (token_repr, bank_indices, weights, topk_idx)


# speedup vs baseline: 1.3700x; 1.0273x over previous
import jax
import jax.numpy as jnp
import numpy as np
from jax.experimental import pallas as pl
from jax.experimental.pallas import tpu as pltpu

D_MODEL = 1024
N_HEADS = 16
D_PHI = 64
D_HEAD = 64
M_SETS = 16
N_CAND = 4
HM = N_HEADS * M_SETS
MIN_TEMP = 0.5
SC_SCALE = 1.0 / np.sqrt(D_PHI)
NEG = -1e30

S_TILE = 2048


def _group_allreduce(x, op):
    lane = jax.lax.broadcasted_iota(jnp.int32, x.shape, 1)
    for k in (1, 2, 4, 8):
        left = pltpu.roll(x, HM - k, axis=1)
        right = pltpu.roll(x, k, axis=1)
        partner = jnp.where((lane & k) == 0, left, right)
        x = op(x, partner)
    return x


def _main_kernel(x_ref, wq_ref, bq_ref, wk_ref, bk_ref, desc_ref, set_ref,
                 t2s_ref, temp_ref,
                 repr_ref, w_ref, i1_ref, i2_ref, bank_ref,
                 kbd_ref, sbd_ref):
    @pl.when(pl.program_id(1) == 0)
    def _prologue():
        k = jnp.dot(desc_ref[0], wk_ref[...],
                    preferred_element_type=jnp.float32) + bk_ref[...]
        rowblocks = []
        for h in range(N_HEADS):
            k_h_t = jnp.transpose(k[:, h * D_PHI:(h + 1) * D_PHI])
            pads = []
            if h > 0:
                pads.append(jnp.zeros((D_PHI, h * M_SETS), jnp.float32))
            pads.append(k_h_t)
            if h < N_HEADS - 1:
                pads.append(jnp.zeros((D_PHI, HM - (h + 1) * M_SETS),
                                      jnp.float32))
            rowblocks.append(jnp.concatenate(pads, axis=1))
        kbd_ref[...] = jnp.concatenate(rowblocks, axis=0)

        set_b = set_ref[0]
        col_h = jax.lax.broadcasted_iota(jnp.int32, (M_SETS, D_MODEL),
                                         1) >> 6
        blocks = [jnp.where(col_h == h, set_b, 0.0) for h in range(N_HEADS)]
        sbd_ref[...] = jnp.concatenate(blocks, axis=0)

    q = jnp.dot(x_ref[0], wq_ref[...],
                preferred_element_type=jnp.float32) + bq_ref[...]
    scores = jnp.dot(q, kbd_ref[...],
                     preferred_element_type=jnp.float32) * SC_SCALE

    s_t = scores.shape[0]
    lane = jax.lax.broadcasted_iota(jnp.int32, (s_t, HM), 1)
    lane_m = lane & 15

    mask = jnp.zeros((s_t, HM), dtype=jnp.bool_)
    for c in range(N_CAND):
        mask = mask | (t2s_ref[:, c:c + 1] == lane_m)
    masked = jnp.where(mask, scores, NEG)

    s_int = jax.lax.bitcast_convert_type(masked, jnp.int32)
    key = jnp.where(s_int < 0, s_int ^ jnp.int32(0x7FFFFFFF), s_int)
    keym = (key & jnp.int32(-16)) | (15 - lane_m)

    k1 = _group_allreduce(keym, jnp.maximum)
    idx1 = 15 - (k1 & 15)
    m1i = jnp.where(k1 < 0, k1 ^ jnp.int32(0x7FFFFFFF), k1)
    m1 = jax.lax.bitcast_convert_type(m1i, jnp.float32)

    keym2 = jnp.where(lane_m == idx1, jnp.int32(-2147483647), keym)
    k2 = _group_allreduce(keym2, jnp.maximum)
    idx2 = 15 - (k2 & 15)

    sel = (lane_m == idx1) | (lane_m == idx2)
    tinv = 1.0 / jnp.maximum(temp_ref[0, 0], MIN_TEMP)
    w_un = jnp.where(sel, jnp.exp((masked - m1) * tinv), 0.0)

    lane_i = jax.lax.broadcasted_iota(jnp.int32, (HM, HM), 0)
    lane_j = jax.lax.broadcasted_iota(jnp.int32, (HM, HM), 1)
    gmat = ((lane_i >> 4) == (lane_j >> 4)).astype(jnp.float32)
    denom = jnp.dot(w_un, gmat, preferred_element_type=jnp.float32,
                    precision=jax.lax.Precision.HIGHEST)
    weights = w_un / denom
    w_ref[0] = weights

    qi = jax.lax.broadcasted_iota(jnp.int32, (HM, M_SETS), 0)
    qj = jax.lax.broadcasted_iota(jnp.int32, (HM, M_SETS), 1)
    qmat = ((qi >> 4) == qj).astype(jnp.float32)
    pmat = ((qi & 15) == qj).astype(jnp.float32)

    start = (lane & 15) == 0
    i1c = jnp.where(start, idx1.astype(jnp.float32), 0.0)
    i2c = jnp.where(start, idx2.astype(jnp.float32), 0.0)
    i1_ref[0] = jnp.dot(i1c, qmat,
                        preferred_element_type=jnp.float32).astype(jnp.int32)
    i2_ref[0] = jnp.dot(i2c, qmat,
                        preferred_element_type=jnp.float32).astype(jnp.int32)

    wsum = jnp.dot(weights, pmat, preferred_element_type=jnp.float32,
                   precision=jax.lax.Precision.HIGHEST)
    bmax = jnp.max(wsum, axis=1, keepdims=True)
    lane16 = jax.lax.broadcasted_iota(jnp.int32, (s_t, M_SETS), 1)
    bank = jnp.min(jnp.where(wsum == bmax, lane16, M_SETS), axis=1,
                   keepdims=True)
    bank_ref[0] = bank

    repr_ref[0] = jnp.dot(weights, sbd_ref[...],
                          preferred_element_type=jnp.float32)


def kernel(token_states, set_states, desc_router, token_to_sets,
           Wq, bq, Wk, bk, temperature):
    b, s, _ = token_states.shape

    n_tiles = s // S_TILE
    token_repr, w_flat, i1, i2, bank = pl.pallas_call(
        _main_kernel,
        grid=(b, n_tiles),
        in_specs=[
            pl.BlockSpec((1, S_TILE, D_MODEL), lambda i, j: (i, j, 0)),
            pl.BlockSpec((D_MODEL, D_MODEL), lambda i, j: (0, 0)),
            pl.BlockSpec((1, D_MODEL), lambda i, j: (0, 0)),
            pl.BlockSpec((D_MODEL, D_MODEL), lambda i, j: (0, 0)),
            pl.BlockSpec((1, D_MODEL), lambda i, j: (0, 0)),
            pl.BlockSpec((1, M_SETS, D_MODEL), lambda i, j: (i, 0, 0)),
            pl.BlockSpec((1, M_SETS, D_MODEL), lambda i, j: (i, 0, 0)),
            pl.BlockSpec((S_TILE, N_CAND), lambda i, j: (j, 0)),
            pl.BlockSpec((1, 1), lambda i, j: (0, 0)),
        ],
        out_specs=[
            pl.BlockSpec((1, S_TILE, D_MODEL), lambda i, j: (i, j, 0)),
            pl.BlockSpec((1, S_TILE, HM), lambda i, j: (i, j, 0)),
            pl.BlockSpec((1, S_TILE, M_SETS), lambda i, j: (i, j, 0)),
            pl.BlockSpec((1, S_TILE, M_SETS), lambda i, j: (i, j, 0)),
            pl.BlockSpec((1, S_TILE, 1), lambda i, j: (i, j, 0)),
        ],
        out_shape=[
            jax.ShapeDtypeStruct((b, s, D_MODEL), jnp.float32),
            jax.ShapeDtypeStruct((b, s, HM), jnp.float32),
            jax.ShapeDtypeStruct((b, s, M_SETS), jnp.int32),
            jax.ShapeDtypeStruct((b, s, M_SETS), jnp.int32),
            jax.ShapeDtypeStruct((b, s, 1), jnp.int32),
        ],
        scratch_shapes=[
            pltpu.VMEM((D_MODEL, HM), jnp.float32),
            pltpu.VMEM((HM, D_MODEL), jnp.float32),
        ],
        compiler_params=pltpu.CompilerParams(
            dimension_semantics=("arbitrary", "arbitrary"),
            vmem_limit_bytes=112 * 1024 * 1024),
    )(token_states, Wq, bq.reshape(1, D_MODEL), Wk, bk.reshape(1, D_MODEL),
      desc_router, set_states, token_to_sets, temperature.reshape(1, 1))

    weights = w_flat.reshape(b, s, N_HEADS, M_SETS).transpose(0, 2, 1, 3)
    topk_idx = jnp.stack([i1, i2], axis=-1).transpose(0, 2, 1, 3)
    bank_indices = bank.reshape(b, s)
    return (token_repr, bank_indices, weights, topk_idx)


# speedup vs baseline: 1.4190x; 1.0358x over previous
import jax
import jax.numpy as jnp
import numpy as np
from jax.experimental import pallas as pl
from jax.experimental.pallas import tpu as pltpu

D_MODEL = 1024
N_HEADS = 16
D_PHI = 64
D_HEAD = 64
M_SETS = 16
N_CAND = 4
HM = N_HEADS * M_SETS
MIN_TEMP = 0.5
SC_SCALE = 1.0 / np.sqrt(D_PHI)
NEG = -1e30

S_TILE = 2048


def _group_allreduce(x, op):
    lane = jax.lax.broadcasted_iota(jnp.int32, x.shape, 1)
    for k in (1, 2, 4, 8):
        left = pltpu.roll(x, HM - k, axis=1)
        right = pltpu.roll(x, k, axis=1)
        partner = jnp.where((lane & k) == 0, left, right)
        x = op(x, partner)
    return x


def _main_kernel(x_ref, wq_ref, bq_ref, wk_ref, bk_ref, desc_ref, set_ref,
                 t2s_ref, temp_ref,
                 repr_ref, w_ref, i1_ref, i2_ref, bank_ref,
                 kbd_ref, sbd_ref):
    @pl.when(pl.program_id(1) == 0)
    def _prologue():
        k = jnp.dot(desc_ref[0], wk_ref[...],
                    preferred_element_type=jnp.float32) + bk_ref[...]
        rowblocks = []
        for h in range(N_HEADS):
            k_h_t = jnp.transpose(k[:, h * D_PHI:(h + 1) * D_PHI])
            pads = []
            if h > 0:
                pads.append(jnp.zeros((D_PHI, h * M_SETS), jnp.float32))
            pads.append(k_h_t)
            if h < N_HEADS - 1:
                pads.append(jnp.zeros((D_PHI, HM - (h + 1) * M_SETS),
                                      jnp.float32))
            rowblocks.append(jnp.concatenate(pads, axis=1))
        kbd_ref[...] = jnp.concatenate(rowblocks, axis=0)

        set_b = set_ref[0]
        col_h = jax.lax.broadcasted_iota(jnp.int32, (M_SETS, D_MODEL),
                                         1) >> 6
        blocks = [jnp.where(col_h == h, set_b, 0.0) for h in range(N_HEADS)]
        sbd_ref[...] = jnp.concatenate(blocks, axis=0)

    q = jnp.dot(x_ref[0], wq_ref[...],
                preferred_element_type=jnp.float32) + bq_ref[...]
    scores = jnp.dot(q, kbd_ref[...],
                     preferred_element_type=jnp.float32) * SC_SCALE

    s_t = scores.shape[0]
    lane = jax.lax.broadcasted_iota(jnp.int32, (s_t, HM), 1)
    lane_m = lane & 15

    mask = jnp.zeros((s_t, HM), dtype=jnp.bool_)
    for c in range(N_CAND):
        mask = mask | (t2s_ref[:, c:c + 1] == lane_m)
    masked = jnp.where(mask, scores, NEG)

    s_int = jax.lax.bitcast_convert_type(masked, jnp.int32)
    key = jnp.where(s_int < 0, s_int ^ jnp.int32(0x7FFFFFFF), s_int)
    keym = (key & jnp.int32(-16)) | (15 - lane_m)

    k1 = _group_allreduce(keym, jnp.maximum)
    idx1 = 15 - (k1 & 15)
    m1i = jnp.where(k1 < 0, k1 ^ jnp.int32(0x7FFFFFFF), k1)
    m1 = jax.lax.bitcast_convert_type(m1i, jnp.float32)

    keym2 = jnp.where(lane_m == idx1, jnp.int32(-2147483647), keym)
    k2 = _group_allreduce(keym2, jnp.maximum)
    idx2 = 15 - (k2 & 15)

    sel = (lane_m == idx1) | (lane_m == idx2)
    tinv = 1.0 / jnp.maximum(temp_ref[0, 0], MIN_TEMP)
    w_un = jnp.where(sel, jnp.exp((masked - m1) * tinv), 0.0)

    lane_i = jax.lax.broadcasted_iota(jnp.int32, (HM, HM), 0)
    lane_j = jax.lax.broadcasted_iota(jnp.int32, (HM, HM), 1)
    gmat = ((lane_i >> 4) == (lane_j >> 4)).astype(jnp.float32)
    denom = jnp.dot(w_un, gmat, preferred_element_type=jnp.float32,
                    precision=jax.lax.Precision.HIGHEST)
    weights = w_un / denom
    w_ref[0] = weights.astype(jnp.bfloat16)

    qi = jax.lax.broadcasted_iota(jnp.int32, (HM, M_SETS), 0)
    qj = jax.lax.broadcasted_iota(jnp.int32, (HM, M_SETS), 1)
    qmat = ((qi >> 4) == qj).astype(jnp.float32)
    pmat = ((qi & 15) == qj).astype(jnp.float32)

    start = (lane & 15) == 0
    i1c = jnp.where(start, idx1.astype(jnp.float32), 0.0)
    i2c = jnp.where(start, idx2.astype(jnp.float32), 0.0)
    i1_ref[0] = jnp.dot(i1c, qmat,
                        preferred_element_type=jnp.float32).astype(jnp.int32)
    i2_ref[0] = jnp.dot(i2c, qmat,
                        preferred_element_type=jnp.float32).astype(jnp.int32)

    wsum = jnp.dot(weights, pmat, preferred_element_type=jnp.float32,
                   precision=jax.lax.Precision.HIGHEST)
    bmax = jnp.max(wsum, axis=1, keepdims=True)
    lane16 = jax.lax.broadcasted_iota(jnp.int32, (s_t, M_SETS), 1)
    bank = jnp.min(jnp.where(wsum == bmax, lane16, M_SETS), axis=1,
                   keepdims=True)
    bank_ref[0] = bank

    repr_ref[0] = jnp.dot(weights, sbd_ref[...],
                          preferred_element_type=jnp.float32)


def kernel(token_states, set_states, desc_router, token_to_sets,
           Wq, bq, Wk, bk, temperature):
    b, s, _ = token_states.shape

    n_tiles = s // S_TILE
    token_repr, w_flat, i1, i2, bank = pl.pallas_call(
        _main_kernel,
        grid=(b, n_tiles),
        in_specs=[
            pl.BlockSpec((1, S_TILE, D_MODEL), lambda i, j: (i, j, 0)),
            pl.BlockSpec((D_MODEL, D_MODEL), lambda i, j: (0, 0)),
            pl.BlockSpec((1, D_MODEL), lambda i, j: (0, 0)),
            pl.BlockSpec((D_MODEL, D_MODEL), lambda i, j: (0, 0)),
            pl.BlockSpec((1, D_MODEL), lambda i, j: (0, 0)),
            pl.BlockSpec((1, M_SETS, D_MODEL), lambda i, j: (i, 0, 0)),
            pl.BlockSpec((1, M_SETS, D_MODEL), lambda i, j: (i, 0, 0)),
            pl.BlockSpec((S_TILE, N_CAND), lambda i, j: (j, 0)),
            pl.BlockSpec((1, 1), lambda i, j: (0, 0)),
        ],
        out_specs=[
            pl.BlockSpec((1, S_TILE, D_MODEL), lambda i, j: (i, j, 0)),
            pl.BlockSpec((1, S_TILE, HM), lambda i, j: (i, j, 0)),
            pl.BlockSpec((1, S_TILE, M_SETS), lambda i, j: (i, j, 0)),
            pl.BlockSpec((1, S_TILE, M_SETS), lambda i, j: (i, j, 0)),
            pl.BlockSpec((1, S_TILE, 1), lambda i, j: (i, j, 0)),
        ],
        out_shape=[
            jax.ShapeDtypeStruct((b, s, D_MODEL), jnp.float32),
            jax.ShapeDtypeStruct((b, s, HM), jnp.bfloat16),
            jax.ShapeDtypeStruct((b, s, M_SETS), jnp.int32),
            jax.ShapeDtypeStruct((b, s, M_SETS), jnp.int32),
            jax.ShapeDtypeStruct((b, s, 1), jnp.int32),
        ],
        scratch_shapes=[
            pltpu.VMEM((D_MODEL, HM), jnp.float32),
            pltpu.VMEM((HM, D_MODEL), jnp.float32),
        ],
        compiler_params=pltpu.CompilerParams(
            dimension_semantics=("arbitrary", "arbitrary"),
            vmem_limit_bytes=112 * 1024 * 1024),
    )(token_states, Wq, bq.reshape(1, D_MODEL), Wk, bk.reshape(1, D_MODEL),
      desc_router, set_states, token_to_sets, temperature.reshape(1, 1))

    weights = w_flat.astype(jnp.float32).reshape(
        b, s, N_HEADS, M_SETS).transpose(0, 2, 1, 3)
    topk_idx = jnp.stack([i1, i2], axis=-1).transpose(0, 2, 1, 3)
    bank_indices = bank.reshape(b, s)
    return (token_repr, bank_indices, weights, topk_idx)
